# Initial kernel scaffold; baseline (speedup 1.0000x reference)
import jax
import jax.numpy as jnp
from jax.experimental import pallas as pl


def kernel(x, edge_index, W0, b0, g0, be0, W1, b1, g1, be1, W2, b2):
    raise NotImplementedError("write your pallas kernel here")


# speedup vs baseline: 6.3002x; 6.3002x over previous
import functools

import jax
import jax.numpy as jnp
from jax import lax
from jax.experimental import pallas as pl
from jax.experimental.pallas import tpu as pltpu
from jax.experimental.pallas import tpu_sc as plsc

_NC = 2
_NS = 16
_LW = 128
_BNS = float(1.0 / (1.0 + 1e-5) ** 0.5)


def _sc_deg(N, NPAD, CHT, ROWS_PER_SC, ZR, NWT, RW):
    mesh = plsc.VectorSubcoreMesh(core_axis_name="c", subcore_axis_name="s")

    @functools.partial(
        pl.kernel,
        out_type=[jax.ShapeDtypeStruct((NPAD,), jnp.float32)] * _NC,
        mesh=mesh,
        scratch_types=[
            pltpu.VMEM((CHT, _LW), jnp.int32),
            pltpu.VMEM((_LW,), jnp.float32),
            pltpu.VMEM((NPAD,), jnp.float32),
            pltpu.VMEM_SHARED((NPAD,), jnp.float32),
        ],
    )
    def k(dstp_hbm, out0_hbm, out1_hbm, dst_slab, ones_v, zbuf, acc):
        c = lax.axis_index("c")
        s = lax.axis_index("s")

        def o16(i, carry):
            ones_v[pl.ds(pl.multiple_of(i * 16, 16), 16)] = jnp.ones(
                (16,), jnp.float32)
            return carry

        lax.fori_loop(0, _LW // 16, o16, 0)

        @pl.when(s == 0)
        def _():
            def z16(i, carry):
                zbuf[pl.ds(pl.multiple_of(i * 16, 16), 16)] = jnp.zeros(
                    (16,), jnp.float32)
                return carry

            lax.fori_loop(0, NPAD // 16, z16, 0)
            pltpu.sync_copy(zbuf, acc)

        base = c * ROWS_PER_SC + s * CHT
        pltpu.sync_copy(dstp_hbm.at[pl.ds(base, CHT)], dst_slab)
        plsc.subcore_barrier()

        def body(j, carry):
            pltpu.sync_copy(ones_v, acc.at[dst_slab.at[j]], add=True)
            return carry

        lax.fori_loop(0, CHT, body, 0)
        plsc.subcore_barrier()

        @pl.when(s == 0)
        def _():
            pltpu.sync_copy(acc, zbuf)

            @pl.when(c == 0)
            def _():
                pltpu.sync_copy(zbuf, out0_hbm)

            @pl.when(c == 1)
            def _():
                pltpu.sync_copy(zbuf, out1_hbm)

    return k


def _sc_spmm(N, D, NPAD, CHT, ROWS_PER_SC, ZR, NWT, RW):
    mesh = plsc.VectorSubcoreMesh(core_axis_name="c", subcore_axis_name="s")

    @functools.partial(
        pl.kernel,
        out_type=jax.ShapeDtypeStruct((_NC, N, D), jnp.float32),
        mesh=mesh,
        scratch_types=[
            pltpu.VMEM((CHT, _LW), jnp.int32),
            pltpu.VMEM((CHT, _LW), jnp.int32),
            pltpu.VMEM((_LW, D), jnp.float32),
            pltpu.VMEM_SHARED((NPAD, D), jnp.float32),
            pltpu.SemaphoreType.DMA,
            pltpu.SemaphoreType.DMA,
        ],
    )
    def k(h_hbm, srcp_hbm, dstp_hbm, zeros_hbm, out_hbm,
          src_slab, dst_slab, rows, acc, gsem, ssem):
        c = lax.axis_index("c")
        s = lax.axis_index("s")
        pltpu.sync_copy(zeros_hbm.at[pl.ds(s * ZR, ZR)], acc.at[pl.ds(s * ZR, ZR)])
        base = c * ROWS_PER_SC + s * CHT
        pltpu.sync_copy(srcp_hbm.at[pl.ds(base, CHT)], src_slab)
        pltpu.sync_copy(dstp_hbm.at[pl.ds(base, CHT)], dst_slab)
        plsc.subcore_barrier()

        def body(j, carry):
            pltpu.async_copy(h_hbm.at[src_slab.at[j]], rows, gsem).wait()
            pltpu.async_copy(rows, acc.at[dst_slab.at[j]], ssem, add=True).wait()
            return carry

        lax.fori_loop(0, CHT, body, 0)
        plsc.subcore_barrier()

        @pl.when(s < NWT)
        def _():
            pltpu.sync_copy(acc.at[pl.ds(s * RW, RW)],
                            out_hbm.at[c, pl.ds(s * RW, RW)])

    return k


def _tc0(degt, x, W0, B):
    N, D = x.shape

    def body(dt_ref, x_ref, w_ref, gp_ref, dv_ref):
        dt = dt_ref[...]
        dinv = lax.rsqrt(1.0 + dt[:, 0:1] + dt[:, 1:2])
        h = jnp.dot(x_ref[...], w_ref[...], preferred_element_type=jnp.float32)
        gp_ref[...] = dinv * h
        dv_ref[...] = jnp.broadcast_to(dinv, (B, D))

    return pl.pallas_call(
        body,
        grid=(N // B,),
        in_specs=[
            pl.BlockSpec((B, 2), lambda i: (i, 0)),
            pl.BlockSpec((B, D), lambda i: (i, 0)),
            pl.BlockSpec((D, D), lambda i: (0, 0)),
        ],
        out_specs=[
            pl.BlockSpec((B, D), lambda i: (i, 0)),
            pl.BlockSpec((B, D), lambda i: (i, 0)),
        ],
        out_shape=[jax.ShapeDtypeStruct((N, D), jnp.float32)] * 2,
    )(degt, x, W0)


def _tc_mid(A, gp, dv, res, W, b, gam, bet, B):
    N, D = gp.shape

    def body(a_ref, gp_ref, dv_ref, res_ref, w_ref, b_ref, g_ref, be_ref,
             z_ref, gn_ref):
        agg = a_ref[0] + a_ref[1] + gp_ref[...]
        u = dv_ref[...] * agg + b_ref[...]
        v = u * (g_ref[...] * _BNS) + be_ref[...]
        z = jnp.where(v >= 0, v, 0.01 * v) + res_ref[...]
        z_ref[...] = z
        gn_ref[...] = dv_ref[...] * jnp.dot(
            z, w_ref[...], preferred_element_type=jnp.float32)

    return pl.pallas_call(
        body,
        grid=(N // B,),
        in_specs=[
            pl.BlockSpec((_NC, B, D), lambda i: (0, i, 0)),
            pl.BlockSpec((B, D), lambda i: (i, 0)),
            pl.BlockSpec((B, D), lambda i: (i, 0)),
            pl.BlockSpec((B, D), lambda i: (i, 0)),
            pl.BlockSpec((D, D), lambda i: (0, 0)),
            pl.BlockSpec((1, D), lambda i: (0, 0)),
            pl.BlockSpec((1, D), lambda i: (0, 0)),
            pl.BlockSpec((1, D), lambda i: (0, 0)),
        ],
        out_specs=[
            pl.BlockSpec((B, D), lambda i: (i, 0)),
            pl.BlockSpec((B, D), lambda i: (i, 0)),
        ],
        out_shape=[jax.ShapeDtypeStruct((N, D), jnp.float32)] * 2,
    )(A, gp, dv, res, W, b, gam, bet)


def _tc_fin(A, gp, dv, b, B):
    N, D = gp.shape

    def body(a_ref, gp_ref, dv_ref, b_ref, o_ref):
        agg = a_ref[0] + a_ref[1] + gp_ref[...]
        o_ref[...] = dv_ref[...] * agg + b_ref[...]

    return pl.pallas_call(
        body,
        grid=(N // B,),
        in_specs=[
            pl.BlockSpec((_NC, B, D), lambda i: (0, i, 0)),
            pl.BlockSpec((B, D), lambda i: (i, 0)),
            pl.BlockSpec((B, D), lambda i: (i, 0)),
            pl.BlockSpec((1, D), lambda i: (0, 0)),
        ],
        out_specs=pl.BlockSpec((B, D), lambda i: (i, 0)),
        out_shape=jax.ShapeDtypeStruct((N, D), jnp.float32),
    )(A, gp, dv, b)


def kernel(x, edge_index, W0, b0, g0, be0, W1, b1, g1, be1, W2, b2):
    N, D = x.shape
    E = edge_index.shape[1]
    src = edge_index[0]
    dst = edge_index[1]

    CHT = -(-E // (_NC * _NS * _LW))
    CHT = -(-CHT // 8) * 8
    EP = _NC * _NS * _LW * CHT
    ROWS = EP // _LW
    ROWS_PER_SC = ROWS // _NC
    pad = EP - E
    srcp = jnp.concatenate([src, jnp.zeros((pad,), src.dtype)]).reshape(ROWS, _LW)
    dstp = jnp.concatenate([dst, jnp.full((pad,), N, dst.dtype)]).reshape(ROWS, _LW)

    NPAD = (N // (8 * _NS) + 1) * (8 * _NS)
    ZR = NPAD // _NS
    NWT = 10
    RW = N // NWT

    zeros2d = jnp.zeros((NPAD, D), jnp.float32)

    B = 1000

    d0, d1 = _sc_deg(N, NPAD, CHT, ROWS_PER_SC, ZR, NWT, RW)(dstp)
    degt = jnp.stack([d0[:N], d1[:N]], axis=1)

    spmm = _sc_spmm(N, D, NPAD, CHT, ROWS_PER_SC, ZR, NWT, RW)

    gp0, dv = _tc0(degt, x, W0, B)
    A0 = spmm(gp0, srcp, dstp, zeros2d)
    z0, gp1 = _tc_mid(A0, gp0, dv, x, W1,
                      b0.reshape(1, D), g0.reshape(1, D), be0.reshape(1, D), B)
    A1 = spmm(gp1, srcp, dstp, zeros2d)
    z1, gp2 = _tc_mid(A1, gp1, dv, z0, W2,
                      b1.reshape(1, D), g1.reshape(1, D), be1.reshape(1, D), B)
    A2 = spmm(gp2, srcp, dstp, zeros2d)
    out = _tc_fin(A2, gp2, dv, b2.reshape(1, D), B)
    return out


# speedup vs baseline: 6.7739x; 1.0752x over previous
import functools

import jax
import jax.numpy as jnp
from jax import lax
from jax.experimental import pallas as pl
from jax.experimental.pallas import tpu as pltpu
from jax.experimental.pallas import tpu_sc as plsc

_NC = 2
_NS = 16
_LW = 128
_BNS = float(1.0 / (1.0 + 1e-5) ** 0.5)


def _sc_deg(N, NPAD, CHT, ROWS_PER_SC, ZR, NWT, RW):
    mesh = plsc.VectorSubcoreMesh(core_axis_name="c", subcore_axis_name="s")

    @functools.partial(
        pl.kernel,
        out_type=[jax.ShapeDtypeStruct((NPAD,), jnp.float32)] * _NC,
        mesh=mesh,
        scratch_types=[
            pltpu.VMEM((CHT, _LW), jnp.int32),
            pltpu.VMEM((_LW,), jnp.float32),
            pltpu.VMEM((NPAD,), jnp.float32),
            pltpu.VMEM_SHARED((NPAD,), jnp.float32),
            pltpu.SemaphoreType.DMA,
        ],
    )
    def k(dstp_hbm, out0_hbm, out1_hbm, dst_slab, ones_v, zbuf, acc, dsem):
        c = lax.axis_index("c")
        s = lax.axis_index("s")

        for kk in range(_LW // 16):
            ones_v[pl.ds(16 * kk, 16)] = jnp.ones((16,), jnp.float32)

        @pl.when(s == 0)
        def _():
            def z16(i, carry):
                zbuf[pl.ds(pl.multiple_of(i * 16, 16), 16)] = jnp.zeros(
                    (16,), jnp.float32)
                return carry

            lax.fori_loop(0, NPAD // 16, z16, 0)
            pltpu.sync_copy(zbuf, acc)

        base = c * ROWS_PER_SC + s * CHT
        pltpu.sync_copy(dstp_hbm.at[pl.ds(base, CHT)], dst_slab)
        plsc.subcore_barrier()

        def fire(j, carry):
            pltpu.async_copy(ones_v, acc.at[dst_slab.at[j]], dsem, add=True)
            return carry

        lax.fori_loop(0, CHT, fire, 0)

        def drain(j, carry):
            pltpu.make_async_copy(ones_v, acc.at[dst_slab.at[0]], dsem).wait()
            return carry

        lax.fori_loop(0, CHT, drain, 0)
        plsc.subcore_barrier()

        @pl.when(s == 0)
        def _():
            pltpu.sync_copy(acc, zbuf)

            @pl.when(c == 0)
            def _():
                pltpu.sync_copy(zbuf, out0_hbm)

            @pl.when(c == 1)
            def _():
                pltpu.sync_copy(zbuf, out1_hbm)

    return k


def _sc_spmm(N, D, NPAD, CHT, ROWS_PER_SC, ZR, NWT, RW):
    mesh = plsc.VectorSubcoreMesh(core_axis_name="c", subcore_axis_name="s")

    SEG = 40
    NSEG = CHT // SEG

    @functools.partial(
        pl.kernel,
        out_type=jax.ShapeDtypeStruct((_NC, N, D), jnp.float32),
        mesh=mesh,
        scratch_types=[
            pltpu.VMEM((SEG, _LW), jnp.int32),
            pltpu.VMEM((SEG, _LW), jnp.int32),
            pltpu.VMEM((2, _LW, D), jnp.float32),
            pltpu.VMEM_SHARED((NPAD, D), jnp.float32),
            pltpu.SemaphoreType.DMA,
            pltpu.SemaphoreType.DMA,
        ],
    )
    def k(h_hbm, srcp_hbm, dstp_hbm, zeros_hbm, out_hbm,
          src_slab, dst_slab, rows, acc, gsem, ssem):
        c = lax.axis_index("c")
        s = lax.axis_index("s")
        pltpu.sync_copy(zeros_hbm.at[pl.ds(s * ZR, ZR)], acc.at[pl.ds(s * ZR, ZR)])
        base = c * ROWS_PER_SC + s * CHT
        plsc.subcore_barrier()

        def g_issue(j, b):
            pltpu.async_copy(h_hbm.at[src_slab.at[j]], rows.at[b], gsem)

        def s_issue(j, b):
            pltpu.async_copy(rows.at[b], acc.at[dst_slab.at[j]], ssem, add=True)

        def g_wait():
            pltpu.make_async_copy(h_hbm.at[src_slab.at[0]], rows.at[0], gsem).wait()

        def s_wait():
            pltpu.make_async_copy(rows.at[0], acc.at[dst_slab.at[0]], ssem).wait()

        def segment(g, carry):
            pltpu.sync_copy(srcp_hbm.at[pl.ds(base + g * SEG, SEG)], src_slab)
            pltpu.sync_copy(dstp_hbm.at[pl.ds(base + g * SEG, SEG)], dst_slab)

            g_issue(0, 0)
            g_wait()
            g_issue(1, 1)
            s_issue(0, 0)

            def body(i, carry2):
                j = 2 * i + 1
                g_wait()
                s_wait()
                g_issue(j + 1, 0)
                s_issue(j, 1)
                g_wait()
                s_wait()
                g_issue(j + 2, 1)
                s_issue(j + 1, 0)
                return carry2

            lax.fori_loop(0, SEG // 2 - 1, body, 0)
            g_wait()
            s_wait()
            s_issue(SEG - 1, 1)
            s_wait()
            return carry

        lax.fori_loop(0, NSEG, segment, 0)
        plsc.subcore_barrier()

        @pl.when(s < NWT)
        def _():
            pltpu.sync_copy(acc.at[pl.ds(s * RW, RW)],
                            out_hbm.at[c, pl.ds(s * RW, RW)])

    return k


def _tc0(degt, x, W0, B):
    N, D = x.shape

    def body(dt_ref, x_ref, w_ref, gp_ref, dv_ref):
        dt = dt_ref[...]
        dinv = lax.rsqrt(1.0 + dt[:, 0:1] + dt[:, 1:2])
        h = jnp.dot(x_ref[...], w_ref[...], preferred_element_type=jnp.float32)
        gp_ref[...] = dinv * h
        dv_ref[...] = jnp.broadcast_to(dinv, (B, D))

    return pl.pallas_call(
        body,
        grid=(N // B,),
        in_specs=[
            pl.BlockSpec((B, 2), lambda i: (i, 0)),
            pl.BlockSpec((B, D), lambda i: (i, 0)),
            pl.BlockSpec((D, D), lambda i: (0, 0)),
        ],
        out_specs=[
            pl.BlockSpec((B, D), lambda i: (i, 0)),
            pl.BlockSpec((B, D), lambda i: (i, 0)),
        ],
        out_shape=[jax.ShapeDtypeStruct((N, D), jnp.float32)] * 2,
    )(degt, x, W0)


def _tc_mid(A, gp, dv, res, W, b, gam, bet, B):
    N, D = gp.shape

    def body(a_ref, gp_ref, dv_ref, res_ref, w_ref, b_ref, g_ref, be_ref,
             z_ref, gn_ref):
        agg = a_ref[0] + a_ref[1] + gp_ref[...]
        u = dv_ref[...] * agg + b_ref[...]
        v = u * (g_ref[...] * _BNS) + be_ref[...]
        z = jnp.where(v >= 0, v, 0.01 * v) + res_ref[...]
        z_ref[...] = z
        gn_ref[...] = dv_ref[...] * jnp.dot(
            z, w_ref[...], preferred_element_type=jnp.float32)

    return pl.pallas_call(
        body,
        grid=(N // B,),
        in_specs=[
            pl.BlockSpec((_NC, B, D), lambda i: (0, i, 0)),
            pl.BlockSpec((B, D), lambda i: (i, 0)),
            pl.BlockSpec((B, D), lambda i: (i, 0)),
            pl.BlockSpec((B, D), lambda i: (i, 0)),
            pl.BlockSpec((D, D), lambda i: (0, 0)),
            pl.BlockSpec((1, D), lambda i: (0, 0)),
            pl.BlockSpec((1, D), lambda i: (0, 0)),
            pl.BlockSpec((1, D), lambda i: (0, 0)),
        ],
        out_specs=[
            pl.BlockSpec((B, D), lambda i: (i, 0)),
            pl.BlockSpec((B, D), lambda i: (i, 0)),
        ],
        out_shape=[jax.ShapeDtypeStruct((N, D), jnp.float32)] * 2,
    )(A, gp, dv, res, W, b, gam, bet)


def _tc_fin(A, gp, dv, b, B):
    N, D = gp.shape

    def body(a_ref, gp_ref, dv_ref, b_ref, o_ref):
        agg = a_ref[0] + a_ref[1] + gp_ref[...]
        o_ref[...] = dv_ref[...] * agg + b_ref[...]

    return pl.pallas_call(
        body,
        grid=(N // B,),
        in_specs=[
            pl.BlockSpec((_NC, B, D), lambda i: (0, i, 0)),
            pl.BlockSpec((B, D), lambda i: (i, 0)),
            pl.BlockSpec((B, D), lambda i: (i, 0)),
            pl.BlockSpec((1, D), lambda i: (0, 0)),
        ],
        out_specs=pl.BlockSpec((B, D), lambda i: (i, 0)),
        out_shape=jax.ShapeDtypeStruct((N, D), jnp.float32),
    )(A, gp, dv, b)


def kernel(x, edge_index, W0, b0, g0, be0, W1, b1, g1, be1, W2, b2):
    N, D = x.shape
    E = edge_index.shape[1]
    src = edge_index[0]
    dst = edge_index[1]

    CHT = -(-E // (_NC * _NS * _LW))
    CHT = -(-CHT // 40) * 40
    EP = _NC * _NS * _LW * CHT
    ROWS = EP // _LW
    ROWS_PER_SC = ROWS // _NC
    pad = EP - E
    srcp = jnp.concatenate([src, jnp.zeros((pad,), src.dtype)]).reshape(ROWS, _LW)
    dstp = jnp.concatenate([dst, jnp.full((pad,), N, dst.dtype)]).reshape(ROWS, _LW)

    NPAD = (N // (8 * _NS) + 1) * (8 * _NS)
    ZR = NPAD // _NS
    NWT = 10
    RW = N // NWT

    zeros2d = jnp.zeros((NPAD, D), jnp.float32)

    B = 1000

    d0, d1 = _sc_deg(N, NPAD, CHT, ROWS_PER_SC, ZR, NWT, RW)(dstp)
    degt = jnp.stack([d0[:N], d1[:N]], axis=1)

    spmm = _sc_spmm(N, D, NPAD, CHT, ROWS_PER_SC, ZR, NWT, RW)

    gp0, dv = _tc0(degt, x, W0, B)
    A0 = spmm(gp0, srcp, dstp, zeros2d)
    z0, gp1 = _tc_mid(A0, gp0, dv, x, W1,
                      b0.reshape(1, D), g0.reshape(1, D), be0.reshape(1, D), B)
    A1 = spmm(gp1, srcp, dstp, zeros2d)
    z1, gp2 = _tc_mid(A1, gp1, dv, z0, W2,
                      b1.reshape(1, D), g1.reshape(1, D), be1.reshape(1, D), B)
    A2 = spmm(gp2, srcp, dstp, zeros2d)
    out = _tc_fin(A2, gp2, dv, b2.reshape(1, D), B)
    return out


# speedup vs baseline: 22.7667x; 3.3610x over previous
import functools

import jax
import jax.numpy as jnp
from jax import lax
from jax.experimental import pallas as pl
from jax.experimental.pallas import tpu as pltpu
from jax.experimental.pallas import tpu_sc as plsc

_NC = 2
_NS = 16
_LW = 128
_BNS = float(1.0 / (1.0 + 1e-5) ** 0.5)


def _sc_deg(N, NPAD, CHT, ROWS_PER_SC, ZR, NWT, RW):
    mesh = plsc.VectorSubcoreMesh(core_axis_name="c", subcore_axis_name="s")

    @functools.partial(
        pl.kernel,
        out_type=[jax.ShapeDtypeStruct((NPAD,), jnp.float32)] * _NC,
        mesh=mesh,
        scratch_types=[
            pltpu.VMEM((CHT, _LW), jnp.int32),
            pltpu.VMEM((_LW,), jnp.float32),
            pltpu.VMEM((NPAD,), jnp.float32),
            pltpu.VMEM_SHARED((NPAD,), jnp.float32),
            pltpu.SemaphoreType.DMA,
        ],
    )
    def k(dstp_hbm, out0_hbm, out1_hbm, dst_slab, ones_v, zbuf, acc, dsem):
        c = lax.axis_index("c")
        s = lax.axis_index("s")

        for kk in range(_LW // 16):
            ones_v[pl.ds(16 * kk, 16)] = jnp.ones((16,), jnp.float32)

        @pl.when(s == 0)
        def _():
            def z16(i, carry):
                zbuf[pl.ds(pl.multiple_of(i * 16, 16), 16)] = jnp.zeros(
                    (16,), jnp.float32)
                return carry

            lax.fori_loop(0, NPAD // 16, z16, 0)
            pltpu.sync_copy(zbuf, acc)

        base = c * ROWS_PER_SC + s * CHT
        pltpu.sync_copy(dstp_hbm.at[pl.ds(base, CHT)], dst_slab)
        plsc.subcore_barrier()

        def fire(j, carry):
            pltpu.async_copy(ones_v, acc.at[dst_slab.at[j]], dsem, add=True)
            return carry

        lax.fori_loop(0, CHT, fire, 0)

        def drain(j, carry):
            pltpu.make_async_copy(ones_v, acc.at[dst_slab.at[0]], dsem).wait()
            return carry

        lax.fori_loop(0, CHT, drain, 0)
        plsc.subcore_barrier()

        @pl.when(s == 0)
        def _():
            pltpu.sync_copy(acc, zbuf)

            @pl.when(c == 0)
            def _():
                pltpu.sync_copy(zbuf, out0_hbm)

            @pl.when(c == 1)
            def _():
                pltpu.sync_copy(zbuf, out1_hbm)

    return k


def _sc_spmm(N, D, NPAD, CHT, ROWS_PER_SC, ZR, NWT, RW):
    mesh = plsc.VectorSubcoreMesh(core_axis_name="c", subcore_axis_name="s")

    SEG = 40
    NSEG = CHT // SEG

    @functools.partial(
        pl.kernel,
        out_type=jax.ShapeDtypeStruct((_NC, N, D), jnp.float32),
        mesh=mesh,
        scratch_types=[
            pltpu.VMEM((SEG, _LW), jnp.int32),
            pltpu.VMEM((SEG, _LW), jnp.int32),
            pltpu.VMEM((2, _LW, D), jnp.float32),
            pltpu.VMEM_SHARED((NPAD, D), jnp.float32),
            pltpu.SemaphoreType.DMA,
            pltpu.SemaphoreType.DMA,
        ],
    )
    def k(h_hbm, srcp_hbm, dstp_hbm, zeros_hbm, out_hbm,
          src_slab, dst_slab, rows, acc, gsem, ssem):
        c = lax.axis_index("c")
        s = lax.axis_index("s")
        pltpu.sync_copy(zeros_hbm.at[pl.ds(s * ZR, ZR)], acc.at[pl.ds(s * ZR, ZR)])
        base = c * ROWS_PER_SC + s * CHT
        plsc.subcore_barrier()

        def g_issue(j, b):
            pltpu.async_copy(h_hbm.at[src_slab.at[j]], rows.at[b], gsem)

        def s_issue(j, b):
            pltpu.async_copy(rows.at[b], acc.at[dst_slab.at[j]], ssem, add=True)

        def g_wait():
            pltpu.make_async_copy(h_hbm.at[src_slab.at[0]], rows.at[0], gsem).wait()

        def s_wait():
            pltpu.make_async_copy(rows.at[0], acc.at[dst_slab.at[0]], ssem).wait()

        def segment(g, carry):
            pltpu.sync_copy(srcp_hbm.at[pl.ds(base + g * SEG, SEG)], src_slab)
            pltpu.sync_copy(dstp_hbm.at[pl.ds(base + g * SEG, SEG)], dst_slab)

            g_issue(0, 0)
            g_wait()
            g_issue(1, 1)
            s_issue(0, 0)

            def body(i, carry2):
                j = 2 * i + 1
                g_wait()
                s_wait()
                g_issue(j + 1, 0)
                s_issue(j, 1)
                g_wait()
                s_wait()
                g_issue(j + 2, 1)
                s_issue(j + 1, 0)
                return carry2

            lax.fori_loop(0, SEG // 2 - 1, body, 0)
            g_wait()
            s_wait()
            s_issue(SEG - 1, 1)
            s_wait()
            return carry

        lax.fori_loop(0, NSEG, segment, 0)
        plsc.subcore_barrier()

        @pl.when(s < NWT)
        def _():
            pltpu.sync_copy(acc.at[pl.ds(s * RW, RW)],
                            out_hbm.at[c, pl.ds(s * RW, RW)])

    return k


def _tc0(degt, x, W0, B):
    N, D = x.shape

    def body(dt_ref, x_ref, w_ref, gp_ref, dv_ref):
        dt = dt_ref[...]
        dinv = lax.rsqrt(1.0 + dt[:, 0:1] + dt[:, 1:2])
        h = jnp.dot(x_ref[...], w_ref[...], preferred_element_type=jnp.float32)
        gp_ref[...] = dinv * h
        dv_ref[...] = jnp.broadcast_to(dinv, (B, D))

    return pl.pallas_call(
        body,
        grid=(N // B,),
        in_specs=[
            pl.BlockSpec((B, 2), lambda i: (i, 0)),
            pl.BlockSpec((B, D), lambda i: (i, 0)),
            pl.BlockSpec((D, D), lambda i: (0, 0)),
        ],
        out_specs=[
            pl.BlockSpec((B, D), lambda i: (i, 0)),
            pl.BlockSpec((B, D), lambda i: (i, 0)),
        ],
        out_shape=[jax.ShapeDtypeStruct((N, D), jnp.float32)] * 2,
    )(degt, x, W0)


def _tc_mid(A, gp, dv, res, W, b, gam, bet, B):
    N, D = gp.shape

    def body(a_ref, gp_ref, dv_ref, res_ref, w_ref, b_ref, g_ref, be_ref,
             z_ref, gn_ref):
        agg = a_ref[0] + a_ref[1] + gp_ref[...]
        u = dv_ref[...] * agg + b_ref[...]
        v = u * (g_ref[...] * _BNS) + be_ref[...]
        z = jnp.where(v >= 0, v, 0.01 * v) + res_ref[...]
        z_ref[...] = z
        gn_ref[...] = dv_ref[...] * jnp.dot(
            z, w_ref[...], preferred_element_type=jnp.float32)

    return pl.pallas_call(
        body,
        grid=(N // B,),
        in_specs=[
            pl.BlockSpec((_NC, B, D), lambda i: (0, i, 0)),
            pl.BlockSpec((B, D), lambda i: (i, 0)),
            pl.BlockSpec((B, D), lambda i: (i, 0)),
            pl.BlockSpec((B, D), lambda i: (i, 0)),
            pl.BlockSpec((D, D), lambda i: (0, 0)),
            pl.BlockSpec((1, D), lambda i: (0, 0)),
            pl.BlockSpec((1, D), lambda i: (0, 0)),
            pl.BlockSpec((1, D), lambda i: (0, 0)),
        ],
        out_specs=[
            pl.BlockSpec((B, D), lambda i: (i, 0)),
            pl.BlockSpec((B, D), lambda i: (i, 0)),
        ],
        out_shape=[jax.ShapeDtypeStruct((N, D), jnp.float32)] * 2,
    )(A, gp, dv, res, W, b, gam, bet)


def _tc_fin(A, gp, dv, b, B):
    N, D = gp.shape

    def body(a_ref, gp_ref, dv_ref, b_ref, o_ref):
        agg = a_ref[0] + a_ref[1] + gp_ref[...]
        o_ref[...] = dv_ref[...] * agg + b_ref[...]

    return pl.pallas_call(
        body,
        grid=(N // B,),
        in_specs=[
            pl.BlockSpec((_NC, B, D), lambda i: (0, i, 0)),
            pl.BlockSpec((B, D), lambda i: (i, 0)),
            pl.BlockSpec((B, D), lambda i: (i, 0)),
            pl.BlockSpec((1, D), lambda i: (0, 0)),
        ],
        out_specs=pl.BlockSpec((B, D), lambda i: (i, 0)),
        out_shape=jax.ShapeDtypeStruct((N, D), jnp.float32),
    )(A, gp, dv, b)


def kernel(x, edge_index, W0, b0, g0, be0, W1, b1, g1, be1, W2, b2):
    N, D = x.shape
    E = edge_index.shape[1]
    src = edge_index[0]
    dst = edge_index[1]

    CHT = -(-E // (_NC * _NS * _LW))
    CHT = -(-CHT // 40) * 40
    EP = _NC * _NS * _LW * CHT
    ROWS = EP // _LW
    ROWS_PER_SC = ROWS // _NC
    pad = EP - E
    NPAD = (N // (8 * _NS) + 1) * (8 * _NS)
    fill = jnp.arange(pad, dtype=src.dtype)
    srcp = jnp.concatenate([src, fill % N]).reshape(ROWS, _LW)
    dstp = jnp.concatenate([dst, N + fill % (NPAD - N)]).reshape(ROWS, _LW)
    ZR = NPAD // _NS
    NWT = 10
    RW = N // NWT

    zeros2d = jnp.zeros((NPAD, D), jnp.float32)

    B = 1000

    d0, d1 = _sc_deg(N, NPAD, CHT, ROWS_PER_SC, ZR, NWT, RW)(dstp)
    degt = jnp.stack([d0[:N], d1[:N]], axis=1)

    spmm = _sc_spmm(N, D, NPAD, CHT, ROWS_PER_SC, ZR, NWT, RW)

    gp0, dv = _tc0(degt, x, W0, B)
    A0 = spmm(gp0, srcp, dstp, zeros2d)
    z0, gp1 = _tc_mid(A0, gp0, dv, x, W1,
                      b0.reshape(1, D), g0.reshape(1, D), be0.reshape(1, D), B)
    A1 = spmm(gp1, srcp, dstp, zeros2d)
    z1, gp2 = _tc_mid(A1, gp1, dv, z0, W2,
                      b1.reshape(1, D), g1.reshape(1, D), be1.reshape(1, D), B)
    A2 = spmm(gp2, srcp, dstp, zeros2d)
    out = _tc_fin(A2, gp2, dv, b2.reshape(1, D), B)
    return out


# speedup vs baseline: 23.2943x; 1.0232x over previous
import functools

import jax
import jax.numpy as jnp
from jax import lax
from jax.experimental import pallas as pl
from jax.experimental.pallas import tpu as pltpu
from jax.experimental.pallas import tpu_sc as plsc

_NC = 2
_NS = 16
_LW = 128
_BNS = float(1.0 / (1.0 + 1e-5) ** 0.5)


def _sc_deg(N, NPAD, CHT, ROWS_PER_SC, ZR, NWT, RW):
    mesh = plsc.VectorSubcoreMesh(core_axis_name="c", subcore_axis_name="s")

    @functools.partial(
        pl.kernel,
        out_type=[jax.ShapeDtypeStruct((NPAD,), jnp.float32)] * _NC,
        mesh=mesh,
        scratch_types=[
            pltpu.VMEM((CHT, _LW), jnp.int32),
            pltpu.VMEM((_LW,), jnp.float32),
            pltpu.VMEM((NPAD,), jnp.float32),
            pltpu.VMEM_SHARED((NPAD,), jnp.float32),
            pltpu.SemaphoreType.DMA,
        ],
    )
    def k(dstp_hbm, out0_hbm, out1_hbm, dst_slab, ones_v, zbuf, acc, dsem):
        c = lax.axis_index("c")
        s = lax.axis_index("s")

        for kk in range(_LW // 16):
            ones_v[pl.ds(16 * kk, 16)] = jnp.ones((16,), jnp.float32)

        @pl.when(s == 0)
        def _():
            def z16(i, carry):
                zbuf[pl.ds(pl.multiple_of(i * 16, 16), 16)] = jnp.zeros(
                    (16,), jnp.float32)
                return carry

            lax.fori_loop(0, NPAD // 16, z16, 0)
            pltpu.sync_copy(zbuf, acc)

        base = c * ROWS_PER_SC + s * CHT
        pltpu.sync_copy(dstp_hbm.at[pl.ds(base, CHT)], dst_slab)
        plsc.subcore_barrier()

        def fire(j, carry):
            pltpu.async_copy(ones_v, acc.at[dst_slab.at[j]], dsem, add=True)
            return carry

        lax.fori_loop(0, CHT, fire, 0)

        def drain(j, carry):
            pltpu.make_async_copy(ones_v, acc.at[dst_slab.at[0]], dsem).wait()
            return carry

        lax.fori_loop(0, CHT, drain, 0)
        plsc.subcore_barrier()

        @pl.when(s == 0)
        def _():
            pltpu.sync_copy(acc, zbuf)

            @pl.when(c == 0)
            def _():
                pltpu.sync_copy(zbuf, out0_hbm)

            @pl.when(c == 1)
            def _():
                pltpu.sync_copy(zbuf, out1_hbm)

    return k


def _sc_spmm(N, D, NPAD, CHT, ROWS_PER_SC, ZR, NWT, RW):
    mesh = plsc.VectorSubcoreMesh(core_axis_name="c", subcore_axis_name="s")

    SEG = 40
    NSEG = CHT // SEG

    @functools.partial(
        pl.kernel,
        out_type=jax.ShapeDtypeStruct((_NC, N, D), jnp.float32),
        mesh=mesh,
        scratch_types=[
            pltpu.VMEM((SEG, _LW), jnp.int32),
            pltpu.VMEM((SEG, _LW), jnp.int32),
            pltpu.VMEM((2, _LW, D), jnp.float32),
            pltpu.VMEM_SHARED((NPAD, D), jnp.float32),
            pltpu.SemaphoreType.DMA,
            pltpu.SemaphoreType.DMA,
        ],
    )
    def k(h_hbm, srcp_hbm, dstp_hbm, out_hbm,
          src_slab, dst_slab, rows, acc, gsem, ssem):
        c = lax.axis_index("c")
        s = lax.axis_index("s")

        def zrow(i, carry):
            for kk in range(D // 16):
                rows[0, i, pl.ds(16 * kk, 16)] = jnp.zeros((16,), jnp.float32)
            return carry

        lax.fori_loop(0, _LW, zrow, 0)
        for q in range(ZR // _LW):
            pltpu.sync_copy(rows.at[0], acc.at[pl.ds(s * ZR + q * _LW, _LW)])
        ZREM = ZR % _LW
        if ZREM:
            pltpu.sync_copy(rows.at[0, pl.ds(0, ZREM)],
                            acc.at[pl.ds(s * ZR + (ZR // _LW) * _LW, ZREM)])

        base = c * ROWS_PER_SC + s * CHT
        plsc.subcore_barrier()

        def g_issue(j, b):
            pltpu.async_copy(h_hbm.at[src_slab.at[j]], rows.at[b], gsem)

        def s_issue(j, b):
            pltpu.async_copy(rows.at[b], acc.at[dst_slab.at[j]], ssem, add=True)

        def g_wait():
            pltpu.make_async_copy(h_hbm.at[src_slab.at[0]], rows.at[0], gsem).wait()

        def s_wait():
            pltpu.make_async_copy(rows.at[0], acc.at[dst_slab.at[0]], ssem).wait()

        def segment(g, carry):
            pltpu.sync_copy(srcp_hbm.at[pl.ds(base + g * SEG, SEG)], src_slab)
            pltpu.sync_copy(dstp_hbm.at[pl.ds(base + g * SEG, SEG)], dst_slab)

            g_issue(0, 0)
            g_wait()
            g_issue(1, 1)
            s_issue(0, 0)

            def body(i, carry2):
                j = 2 * i + 1
                g_wait()
                s_wait()
                g_issue(j + 1, 0)
                s_issue(j, 1)
                g_wait()
                s_wait()
                g_issue(j + 2, 1)
                s_issue(j + 1, 0)
                return carry2

            lax.fori_loop(0, SEG // 2 - 1, body, 0)
            g_wait()
            s_wait()
            s_issue(SEG - 1, 1)
            s_wait()
            return carry

        lax.fori_loop(0, NSEG, segment, 0)
        plsc.subcore_barrier()

        @pl.when(s < NWT)
        def _():
            pltpu.sync_copy(acc.at[pl.ds(s * RW, RW)],
                            out_hbm.at[c, pl.ds(s * RW, RW)])

    return k


def _tc0(degt, x, W0, B):
    N, D = x.shape

    def body(dt_ref, x_ref, w_ref, gp_ref, dv_ref):
        dt = dt_ref[...]
        dinv = lax.rsqrt(1.0 + dt[:, 0:1] + dt[:, 1:2])
        h = jnp.dot(x_ref[...], w_ref[...], preferred_element_type=jnp.float32)
        gp_ref[...] = dinv * h
        dv_ref[...] = jnp.broadcast_to(dinv, (B, D))

    return pl.pallas_call(
        body,
        grid=(N // B,),
        in_specs=[
            pl.BlockSpec((B, 2), lambda i: (i, 0)),
            pl.BlockSpec((B, D), lambda i: (i, 0)),
            pl.BlockSpec((D, D), lambda i: (0, 0)),
        ],
        out_specs=[
            pl.BlockSpec((B, D), lambda i: (i, 0)),
            pl.BlockSpec((B, D), lambda i: (i, 0)),
        ],
        out_shape=[jax.ShapeDtypeStruct((N, D), jnp.float32)] * 2,
    )(degt, x, W0)


def _tc_mid(A, gp, dv, res, W, b, gam, bet, B):
    N, D = gp.shape

    def body(a_ref, gp_ref, dv_ref, res_ref, w_ref, b_ref, g_ref, be_ref,
             z_ref, gn_ref):
        agg = a_ref[0] + a_ref[1] + gp_ref[...]
        u = dv_ref[...] * agg + b_ref[...]
        v = u * (g_ref[...] * _BNS) + be_ref[...]
        z = jnp.where(v >= 0, v, 0.01 * v) + res_ref[...]
        z_ref[...] = z
        gn_ref[...] = dv_ref[...] * jnp.dot(
            z, w_ref[...], preferred_element_type=jnp.float32)

    return pl.pallas_call(
        body,
        grid=(N // B,),
        in_specs=[
            pl.BlockSpec((_NC, B, D), lambda i: (0, i, 0)),
            pl.BlockSpec((B, D), lambda i: (i, 0)),
            pl.BlockSpec((B, D), lambda i: (i, 0)),
            pl.BlockSpec((B, D), lambda i: (i, 0)),
            pl.BlockSpec((D, D), lambda i: (0, 0)),
            pl.BlockSpec((1, D), lambda i: (0, 0)),
            pl.BlockSpec((1, D), lambda i: (0, 0)),
            pl.BlockSpec((1, D), lambda i: (0, 0)),
        ],
        out_specs=[
            pl.BlockSpec((B, D), lambda i: (i, 0)),
            pl.BlockSpec((B, D), lambda i: (i, 0)),
        ],
        out_shape=[jax.ShapeDtypeStruct((N, D), jnp.float32)] * 2,
    )(A, gp, dv, res, W, b, gam, bet)


def _tc_fin(A, gp, dv, b, B):
    N, D = gp.shape

    def body(a_ref, gp_ref, dv_ref, b_ref, o_ref):
        agg = a_ref[0] + a_ref[1] + gp_ref[...]
        o_ref[...] = dv_ref[...] * agg + b_ref[...]

    return pl.pallas_call(
        body,
        grid=(N // B,),
        in_specs=[
            pl.BlockSpec((_NC, B, D), lambda i: (0, i, 0)),
            pl.BlockSpec((B, D), lambda i: (i, 0)),
            pl.BlockSpec((B, D), lambda i: (i, 0)),
            pl.BlockSpec((1, D), lambda i: (0, 0)),
        ],
        out_specs=pl.BlockSpec((B, D), lambda i: (i, 0)),
        out_shape=jax.ShapeDtypeStruct((N, D), jnp.float32),
    )(A, gp, dv, b)


def kernel(x, edge_index, W0, b0, g0, be0, W1, b1, g1, be1, W2, b2):
    N, D = x.shape
    E = edge_index.shape[1]
    src = edge_index[0]
    dst = edge_index[1]

    CHT = -(-E // (_NC * _NS * _LW))
    CHT = -(-CHT // 40) * 40
    EP = _NC * _NS * _LW * CHT
    ROWS = EP // _LW
    ROWS_PER_SC = ROWS // _NC
    pad = EP - E
    NPAD = (N // (8 * _NS) + 1) * (8 * _NS)
    fill = jnp.arange(pad, dtype=src.dtype)
    srcp = jnp.concatenate([src, fill % N]).reshape(ROWS, _LW)
    dstp = jnp.concatenate([dst, N + fill % (NPAD - N)]).reshape(ROWS, _LW)
    ZR = NPAD // _NS
    NWT = 10
    RW = N // NWT

    B = 1000

    d0, d1 = _sc_deg(N, NPAD, CHT, ROWS_PER_SC, ZR, NWT, RW)(dstp)
    degt = jnp.stack([d0[:N], d1[:N]], axis=1)

    spmm = _sc_spmm(N, D, NPAD, CHT, ROWS_PER_SC, ZR, NWT, RW)

    gp0, dv = _tc0(degt, x, W0, B)
    A0 = spmm(gp0, srcp, dstp)
    z0, gp1 = _tc_mid(A0, gp0, dv, x, W1,
                      b0.reshape(1, D), g0.reshape(1, D), be0.reshape(1, D), B)
    A1 = spmm(gp1, srcp, dstp)
    z1, gp2 = _tc_mid(A1, gp1, dv, z0, W2,
                      b1.reshape(1, D), g1.reshape(1, D), be1.reshape(1, D), B)
    A2 = spmm(gp2, srcp, dstp)
    out = _tc_fin(A2, gp2, dv, b2.reshape(1, D), B)
    return out
